# baseline (speedup 1.0000x reference)
import functools

import jax
import jax.numpy as jnp
from jax import lax
from jax.experimental import pallas as pl
from jax.experimental.pallas import tpu as pltpu
from jax.experimental.pallas import tpu_sc as plsc

VOCAB = 1000000
SENT_LEN = 200
DIM = 64
BATCH = 4096

NW = 32
CHUNK = 128
NCHUNK = SENT_LEN
NBUF = 4
LOOKAHEAD = 3
NLANE = 16
DTILE = DIM // 8

_mesh = plsc.VectorSubcoreMesh(core_axis_name="c", subcore_axis_name="s")


VTILES = VOCAB // 128
VREM = VOCAB - VTILES * 128
TBLK = VTILES // NW
RNBUF = 4
RLOOK = 3
RPITCH = 129


@functools.partial(
    pl.kernel,
    out_type=jax.ShapeDtypeStruct((VOCAB // 2, 128), jnp.float32),
    mesh=_mesh,
    scratch_types=[pltpu.VMEM((DIM, 128), jnp.float32) for _ in range(RNBUF)]
    + [pltpu.VMEM((VREM, VREM), jnp.float32)]
    + [pltpu.VMEM((DIM * RPITCH,), jnp.float32)]
    + [pltpu.VMEM((DIM, 128), jnp.float32) for _ in range(RNBUF)]
    + [pltpu.SemaphoreType.DMA for _ in range(2 * RNBUF)],
    compiler_params=pltpu.CompilerParams(
        use_tc_tiling_on_sc=True, needs_layout_passes=False),
)
def _table_repack(tokt_hbm, tail_hbm, out_hbm, *bufs_sems):
    ibuf = list(bufs_sems[:RNBUF])
    tibuf = bufs_sems[RNBUF]
    mid = bufs_sems[RNBUF + 1]
    sbuf = list(bufs_sems[RNBUF + 2:2 * RNBUF + 2])
    isem = list(bufs_sems[2 * RNBUF + 2:3 * RNBUF + 2])
    osem = list(bufs_sems[3 * RNBUF + 2:])

    wid = lax.axis_index("s") * 2 + lax.axis_index("c")
    iota = lax.iota(jnp.int32, NLANE)
    s1base = [iota + 16 * k for k in range(128 // NLANE)]
    s2base = [(lax.rem(jnp.int32(16 * j), jnp.int32(DIM)) + iota) * RPITCH
              + (16 * j) // DIM
              for j in range(128 // NLANE)]

    def blk(t):
        return wid + NW * t

    def issue_in(t, b):
        pltpu.async_copy(tokt_hbm.at[:, pl.ds(blk(t) * 128, 128)],
                         ibuf[b], isem[b])

    def wait_in(b):
        pltpu.make_async_copy(tokt_hbm.at[:, pl.ds(0, 128)], ibuf[b],
                              isem[b]).wait()

    def issue_out(t, b):
        r0 = blk(t) * DIM
        pltpu.async_copy(sbuf[b], out_hbm.at[pl.ds(r0, DIM)], osem[b])

    def wait_out(b):
        pltpu.make_async_copy(sbuf[b], out_hbm.at[pl.ds(0, DIM)],
                              osem[b]).wait()

    def transpose(b, nk=128 // NLANE):
        src = ibuf[b]
        dst = sbuf[b]

        @plsc.parallel_loop(0, DIM, unroll=2)
        def _(d):
            dp = d * RPITCH
            for k in range(nk):
                vals = src[d, pl.ds(k * NLANE, NLANE)]
                plsc.store_scatter(mid, [s1base[k] + dp], vals)

        @plsc.parallel_loop(0, DIM, unroll=2)
        def _(m):
            m2 = m * 2
            for j in range(nk):
                vals = plsc.load_gather(mid, [s2base[j] + m2])
                dst[m, pl.ds(j * NLANE, NLANE)] = vals

    def stage(t, b, *, wait_o, issue_i):
        wait_in(b)
        if wait_o:
            wait_out(b)
        transpose(b)
        issue_out(t, b)
        if issue_i:
            issue_in(t + RLOOK, (b + RLOOK) % RNBUF)

    for t in range(RLOOK):
        issue_in(t, t)
    for t in range(RNBUF):
        stage(t, t, wait_o=False, issue_i=True)

    def outer(t3, carry):
        for b in range(RNBUF):
            stage(t3 * RNBUF + b, b, wait_o=True, issue_i=True)
        return carry

    lax.fori_loop(1, TBLK // RNBUF - 1, outer, 0)

    for t in range(RNBUF * (TBLK // RNBUF - 1), TBLK):
        stage(t, t % RNBUF, wait_o=True, issue_i=(t + RLOOK < TBLK))
    for b in range(RNBUF):
        wait_out(b)

    @pl.when(wid < VTILES - NW * TBLK)
    def _():
        v0 = (NW * TBLK + wid) * 128
        pltpu.sync_copy(tokt_hbm.at[:, pl.ds(v0, 128)], ibuf[0])
        transpose(0)
        r0 = (NW * TBLK + wid) * DIM
        pltpu.sync_copy(sbuf[0].at[:, pl.ds(0, 128)],
                        out_hbm.at[pl.ds(r0, DIM)])

    @pl.when(wid == NW - 1)
    def _():
        pltpu.sync_copy(tail_hbm, tibuf)
        dst = sbuf[1]

        @plsc.parallel_loop(0, DIM, unroll=2)
        def _(d):
            dp = d * RPITCH
            for k in range(VREM // NLANE):
                vals = tibuf[d, pl.ds(k * NLANE, NLANE)]
                plsc.store_scatter(mid, [s1base[k] + dp], vals)

        @plsc.parallel_loop(0, VREM // 2, unroll=2)
        def _(m):
            m2 = m * 2
            for j in range(128 // NLANE):
                vals = plsc.load_gather(mid, [s2base[j] + m2])
                dst[m, pl.ds(j * NLANE, NLANE)] = vals

        r0 = VTILES * DIM
        pltpu.sync_copy(sbuf[1].at[pl.ds(0, VREM // 2), pl.ds(0, 128)],
                        out_hbm.at[pl.ds(r0, VREM // 2)])


@functools.partial(
    pl.kernel,
    out_type=jax.ShapeDtypeStruct((SENT_LEN, DTILE, NW, 8, 128), jnp.float32),
    mesh=_mesh,
    scratch_types=[
        pltpu.VMEM((NCHUNK, CHUNK), jnp.int32),
        pltpu.VMEM((SENT_LEN, DIM), jnp.float32),
    ]
    + [pltpu.VMEM((CHUNK, DIM), jnp.float32) for _ in range(NBUF)]
    + [pltpu.VMEM((DTILE, 8, 129), jnp.float32) for _ in range(NBUF)]
    + [pltpu.SemaphoreType.DMA for _ in range(2 * NBUF)],
    compiler_params=pltpu.CompilerParams(
        use_tc_tiling_on_sc=False, needs_layout_passes=False),
)
def _emb_lookup(idx_hbm, tok_hbm, pos_hbm, out_hbm, idx_v, pos_v, *bufs_sems):
    gbuf = list(bufs_sems[:NBUF])
    tbuf = list(bufs_sems[NBUF:2 * NBUF])
    gsem = list(bufs_sems[2 * NBUF:3 * NBUF])
    ssem = list(bufs_sems[3 * NBUF:])

    wid = lax.axis_index("s") * 2 + lax.axis_index("c")

    pltpu.sync_copy(idx_hbm.at[:, pl.ds(wid * CHUNK, CHUNK)], idx_v)
    pltpu.sync_copy(pos_hbm, pos_v)

    iota = lax.iota(jnp.int32, NLANE)

    def issue_gather(c, b):
        pltpu.async_copy(tok_hbm.at[idx_v.at[c]], gbuf[b], gsem[b])

    def wait_gather(b):
        pltpu.make_async_copy(tok_hbm.at[pl.ds(0, CHUNK)], gbuf[b], gsem[b]).wait()

    def issue_scatter(c, b):
        pltpu.async_copy(tbuf[b].at[:, :, pl.ds(0, 128)],
                         out_hbm.at[c, :, wid], ssem[b])

    def wait_scatter(b):
        pltpu.make_async_copy(tbuf[b].at[:, :, pl.ds(0, 128)],
                              out_hbm.at[0, :, 0], ssem[b]).wait()

    dr_vecs = [lax.shift_right_logical(iota + j * NLANE, 3)
               for j in range(DIM // NLANE)]
    di_vecs = [lax.bitwise_and(iota + j * NLANE, 7)
               for j in range(DIM // NLANE)]

    def transpose_add(c, b):
        src = gbuf[b]
        dst = tbuf[b]
        posv = [pos_v[c, pl.ds(j * NLANE, NLANE)] for j in range(DIM // NLANE)]

        @plsc.parallel_loop(0, CHUNK, unroll=2)
        def _(bb):
            bsplat = jnp.full((NLANE,), bb, jnp.int32)
            for j in range(DIM // NLANE):
                vals = src[bb, pl.ds(j * NLANE, NLANE)] + posv[j]
                plsc.store_scatter(dst, [dr_vecs[j], di_vecs[j], bsplat], vals)

    def stage(c, b, *, wait_sc, issue_g):
        wait_gather(b)
        if wait_sc:
            wait_scatter(b)
        transpose_add(c, b)
        issue_scatter(c, b)
        if issue_g:
            issue_gather(c + LOOKAHEAD, (b + LOOKAHEAD) % NBUF)

    for c in range(LOOKAHEAD):
        issue_gather(c, c)

    for b in range(NBUF):
        stage(b, b, wait_sc=False, issue_g=True)

    def outer(c4, carry):
        for b in range(NBUF):
            stage(c4 * NBUF + b, b, wait_sc=True, issue_g=True)
        return carry

    lax.fori_loop(1, NCHUNK // NBUF - 1, outer, 0)

    for b in range(NBUF):
        c = NCHUNK - NBUF + b
        stage(c, b, wait_sc=True, issue_g=(c + LOOKAHEAD < NCHUNK))

    for b in range(NBUF):
        wait_scatter(b)


def kernel(inputs, token_table, position_table):
    idx_t = inputs.T.astype(jnp.int32)
    tok_rm = _table_repack(
        token_table.T, token_table[VTILES * 128:].T).reshape(VOCAB, DIM)
    out5 = _emb_lookup(idx_t, tok_rm, position_table)
    return out5.transpose(2, 4, 0, 1, 3).reshape(BATCH, SENT_LEN, DIM)


# speedup vs baseline: 1.0113x; 1.0113x over previous
import functools

import jax
import jax.numpy as jnp
from jax import lax
from jax.experimental import pallas as pl
from jax.experimental.pallas import tpu as pltpu
from jax.experimental.pallas import tpu_sc as plsc

VOCAB = 1000000
SENT_LEN = 200
DIM = 64
BATCH = 4096

NW = 32
CHUNK = 128
NCHUNK = SENT_LEN
NBUF = 5
LOOKAHEAD = 4
NLANE = 16
DTILE = DIM // 8

_mesh = plsc.VectorSubcoreMesh(core_axis_name="c", subcore_axis_name="s")


VTILES = VOCAB // 128
VREM = VOCAB - VTILES * 128
TBLK = VTILES // NW
RNBUF = 4
RLOOK = 3
RPITCH = 129


@functools.partial(
    pl.kernel,
    out_type=jax.ShapeDtypeStruct((VOCAB // 2, 128), jnp.float32),
    mesh=_mesh,
    scratch_types=[pltpu.VMEM((DIM, 128), jnp.float32) for _ in range(RNBUF)]
    + [pltpu.VMEM((VREM, VREM), jnp.float32)]
    + [pltpu.VMEM((DIM * RPITCH,), jnp.float32)]
    + [pltpu.VMEM((DIM, 128), jnp.float32) for _ in range(RNBUF)]
    + [pltpu.SemaphoreType.DMA for _ in range(2 * RNBUF)],
    compiler_params=pltpu.CompilerParams(
        use_tc_tiling_on_sc=True, needs_layout_passes=False),
)
def _table_repack(tokt_hbm, tail_hbm, out_hbm, *bufs_sems):
    ibuf = list(bufs_sems[:RNBUF])
    tibuf = bufs_sems[RNBUF]
    mid = bufs_sems[RNBUF + 1]
    sbuf = list(bufs_sems[RNBUF + 2:2 * RNBUF + 2])
    isem = list(bufs_sems[2 * RNBUF + 2:3 * RNBUF + 2])
    osem = list(bufs_sems[3 * RNBUF + 2:])

    wid = lax.axis_index("s") * 2 + lax.axis_index("c")
    iota = lax.iota(jnp.int32, NLANE)
    s1base = [iota + 16 * k for k in range(128 // NLANE)]
    s2base = [(lax.rem(jnp.int32(16 * j), jnp.int32(DIM)) + iota) * RPITCH
              + (16 * j) // DIM
              for j in range(128 // NLANE)]

    def blk(t):
        return wid + NW * t

    def issue_in(t, b):
        pltpu.async_copy(tokt_hbm.at[:, pl.ds(blk(t) * 128, 128)],
                         ibuf[b], isem[b])

    def wait_in(b):
        pltpu.make_async_copy(tokt_hbm.at[:, pl.ds(0, 128)], ibuf[b],
                              isem[b]).wait()

    def issue_out(t, b):
        r0 = blk(t) * DIM
        pltpu.async_copy(sbuf[b], out_hbm.at[pl.ds(r0, DIM)], osem[b])

    def wait_out(b):
        pltpu.make_async_copy(sbuf[b], out_hbm.at[pl.ds(0, DIM)],
                              osem[b]).wait()

    def transpose(b, nk=128 // NLANE):
        src = ibuf[b]
        dst = sbuf[b]

        @plsc.parallel_loop(0, DIM, unroll=2)
        def _(d):
            dp = d * RPITCH
            for k in range(nk):
                vals = src[d, pl.ds(k * NLANE, NLANE)]
                plsc.store_scatter(mid, [s1base[k] + dp], vals)

        @plsc.parallel_loop(0, DIM, unroll=2)
        def _(m):
            m2 = m * 2
            for j in range(nk):
                vals = plsc.load_gather(mid, [s2base[j] + m2])
                dst[m, pl.ds(j * NLANE, NLANE)] = vals

    def stage(t, b, *, wait_o, issue_i):
        wait_in(b)
        if wait_o:
            wait_out(b)
        transpose(b)
        issue_out(t, b)
        if issue_i:
            issue_in(t + RLOOK, (b + RLOOK) % RNBUF)

    for t in range(RLOOK):
        issue_in(t, t)
    for t in range(RNBUF):
        stage(t, t, wait_o=False, issue_i=True)

    def outer(t3, carry):
        for b in range(RNBUF):
            stage(t3 * RNBUF + b, b, wait_o=True, issue_i=True)
        return carry

    lax.fori_loop(1, TBLK // RNBUF - 1, outer, 0)

    for t in range(RNBUF * (TBLK // RNBUF - 1), TBLK):
        stage(t, t % RNBUF, wait_o=True, issue_i=(t + RLOOK < TBLK))
    for b in range(RNBUF):
        wait_out(b)

    @pl.when(wid < VTILES - NW * TBLK)
    def _():
        v0 = (NW * TBLK + wid) * 128
        pltpu.sync_copy(tokt_hbm.at[:, pl.ds(v0, 128)], ibuf[0])
        transpose(0)
        r0 = (NW * TBLK + wid) * DIM
        pltpu.sync_copy(sbuf[0].at[:, pl.ds(0, 128)],
                        out_hbm.at[pl.ds(r0, DIM)])

    @pl.when(wid == NW - 1)
    def _():
        pltpu.sync_copy(tail_hbm, tibuf)
        dst = sbuf[1]

        @plsc.parallel_loop(0, DIM, unroll=2)
        def _(d):
            dp = d * RPITCH
            for k in range(VREM // NLANE):
                vals = tibuf[d, pl.ds(k * NLANE, NLANE)]
                plsc.store_scatter(mid, [s1base[k] + dp], vals)

        @plsc.parallel_loop(0, VREM // 2, unroll=2)
        def _(m):
            m2 = m * 2
            for j in range(128 // NLANE):
                vals = plsc.load_gather(mid, [s2base[j] + m2])
                dst[m, pl.ds(j * NLANE, NLANE)] = vals

        r0 = VTILES * DIM
        pltpu.sync_copy(sbuf[1].at[pl.ds(0, VREM // 2), pl.ds(0, 128)],
                        out_hbm.at[pl.ds(r0, VREM // 2)])


@functools.partial(
    pl.kernel,
    out_type=jax.ShapeDtypeStruct((SENT_LEN, DTILE, NW, 8, 128), jnp.float32),
    mesh=_mesh,
    scratch_types=[
        pltpu.VMEM((NCHUNK, CHUNK), jnp.int32),
        pltpu.VMEM((SENT_LEN, DIM), jnp.float32),
    ]
    + [pltpu.VMEM((CHUNK, DIM), jnp.float32) for _ in range(NBUF)]
    + [pltpu.VMEM((DTILE, 8, 129), jnp.float32) for _ in range(NBUF)]
    + [pltpu.SemaphoreType.DMA for _ in range(2 * NBUF)],
    compiler_params=pltpu.CompilerParams(
        use_tc_tiling_on_sc=False, needs_layout_passes=False),
)
def _emb_lookup(idx_hbm, tok_hbm, pos_hbm, out_hbm, idx_v, pos_v, *bufs_sems):
    gbuf = list(bufs_sems[:NBUF])
    tbuf = list(bufs_sems[NBUF:2 * NBUF])
    gsem = list(bufs_sems[2 * NBUF:3 * NBUF])
    ssem = list(bufs_sems[3 * NBUF:])

    wid = lax.axis_index("s") * 2 + lax.axis_index("c")

    pltpu.sync_copy(idx_hbm.at[:, pl.ds(wid * CHUNK, CHUNK)], idx_v)
    pltpu.sync_copy(pos_hbm, pos_v)

    iota = lax.iota(jnp.int32, NLANE)

    def issue_gather(c, b):
        pltpu.async_copy(tok_hbm.at[idx_v.at[c]], gbuf[b], gsem[b])

    def wait_gather(b):
        pltpu.make_async_copy(tok_hbm.at[pl.ds(0, CHUNK)], gbuf[b], gsem[b]).wait()

    def issue_scatter(c, b):
        pltpu.async_copy(tbuf[b].at[:, :, pl.ds(0, 128)],
                         out_hbm.at[c, :, wid], ssem[b])

    def wait_scatter(b):
        pltpu.make_async_copy(tbuf[b].at[:, :, pl.ds(0, 128)],
                              out_hbm.at[0, :, 0], ssem[b]).wait()

    dr_vecs = [lax.shift_right_logical(iota + j * NLANE, 3)
               for j in range(DIM // NLANE)]
    di_vecs = [lax.bitwise_and(iota + j * NLANE, 7)
               for j in range(DIM // NLANE)]

    def transpose_add(c, b):
        src = gbuf[b]
        dst = tbuf[b]
        posv = [pos_v[c, pl.ds(j * NLANE, NLANE)] for j in range(DIM // NLANE)]

        @plsc.parallel_loop(0, CHUNK, unroll=2)
        def _(bb):
            bsplat = jnp.full((NLANE,), bb, jnp.int32)
            for j in range(DIM // NLANE):
                vals = src[bb, pl.ds(j * NLANE, NLANE)] + posv[j]
                plsc.store_scatter(dst, [dr_vecs[j], di_vecs[j], bsplat], vals)

    def stage(c, b, *, wait_sc, issue_g):
        wait_gather(b)
        if wait_sc:
            wait_scatter(b)
        transpose_add(c, b)
        issue_scatter(c, b)
        if issue_g:
            issue_gather(c + LOOKAHEAD, (b + LOOKAHEAD) % NBUF)

    for c in range(LOOKAHEAD):
        issue_gather(c, c)

    for b in range(NBUF):
        stage(b, b, wait_sc=False, issue_g=True)

    def outer(c4, carry):
        for b in range(NBUF):
            stage(c4 * NBUF + b, b, wait_sc=True, issue_g=True)
        return carry

    lax.fori_loop(1, NCHUNK // NBUF - 1, outer, 0)

    for b in range(NBUF):
        c = NCHUNK - NBUF + b
        stage(c, b, wait_sc=True, issue_g=(c + LOOKAHEAD < NCHUNK))

    for b in range(NBUF):
        wait_scatter(b)


def kernel(inputs, token_table, position_table):
    idx_t = inputs.T.astype(jnp.int32)
    tok_rm = _table_repack(
        token_table.T, token_table[VTILES * 128:].T).reshape(VOCAB, DIM)
    out5 = _emb_lookup(idx_t, tok_rm, position_table)
    return out5.transpose(2, 4, 0, 1, 3).reshape(BATCH, SENT_LEN, DIM)


# speedup vs baseline: 1.0266x; 1.0151x over previous
import functools

import jax
import jax.numpy as jnp
from jax import lax
from jax.experimental import pallas as pl
from jax.experimental.pallas import tpu as pltpu
from jax.experimental.pallas import tpu_sc as plsc

VOCAB = 1000000
SENT_LEN = 200
DIM = 64
BATCH = 4096

NW = 32
CHUNK = 128
NCHUNK = SENT_LEN
NBUF = 5
LOOKAHEAD = 4
NLANE = 16
DTILE = DIM // 8

_mesh = plsc.VectorSubcoreMesh(core_axis_name="c", subcore_axis_name="s")


VTILES = VOCAB // 128
VREM = VOCAB - VTILES * 128
TBLK = VTILES // NW
RNBUF = 5
RLOOK = 4
RPITCH = 129


@functools.partial(
    pl.kernel,
    out_type=jax.ShapeDtypeStruct((VOCAB // 2, 128), jnp.float32),
    mesh=_mesh,
    scratch_types=[pltpu.VMEM((DIM, 128), jnp.float32) for _ in range(RNBUF)]
    + [pltpu.VMEM((VREM, VREM), jnp.float32)]
    + [pltpu.VMEM((DIM * RPITCH,), jnp.float32)]
    + [pltpu.VMEM((DIM, 128), jnp.float32) for _ in range(RNBUF)]
    + [pltpu.SemaphoreType.DMA for _ in range(2 * RNBUF)],
    compiler_params=pltpu.CompilerParams(
        use_tc_tiling_on_sc=True, needs_layout_passes=False),
)
def _table_repack(tokt_hbm, tail_hbm, out_hbm, *bufs_sems):
    ibuf = list(bufs_sems[:RNBUF])
    tibuf = bufs_sems[RNBUF]
    mid = bufs_sems[RNBUF + 1]
    sbuf = list(bufs_sems[RNBUF + 2:2 * RNBUF + 2])
    isem = list(bufs_sems[2 * RNBUF + 2:3 * RNBUF + 2])
    osem = list(bufs_sems[3 * RNBUF + 2:])

    wid = lax.axis_index("s") * 2 + lax.axis_index("c")
    iota = lax.iota(jnp.int32, NLANE)
    s1base = [iota + 16 * k for k in range(128 // NLANE)]
    s2base = [(lax.rem(jnp.int32(16 * j), jnp.int32(DIM)) + iota) * RPITCH
              + (16 * j) // DIM
              for j in range(128 // NLANE)]

    def blk(t):
        return wid + NW * t

    def issue_in(t, b):
        pltpu.async_copy(tokt_hbm.at[:, pl.ds(blk(t) * 128, 128)],
                         ibuf[b], isem[b])

    def wait_in(b):
        pltpu.make_async_copy(tokt_hbm.at[:, pl.ds(0, 128)], ibuf[b],
                              isem[b]).wait()

    def issue_out(t, b):
        r0 = blk(t) * DIM
        pltpu.async_copy(sbuf[b], out_hbm.at[pl.ds(r0, DIM)], osem[b])

    def wait_out(b):
        pltpu.make_async_copy(sbuf[b], out_hbm.at[pl.ds(0, DIM)],
                              osem[b]).wait()

    def transpose(b, nk=128 // NLANE):
        src = ibuf[b]
        dst = sbuf[b]

        @plsc.parallel_loop(0, DIM, unroll=2)
        def _(d):
            dp = d * RPITCH
            for k in range(nk):
                vals = src[d, pl.ds(k * NLANE, NLANE)]
                plsc.store_scatter(mid, [s1base[k] + dp], vals)

        @plsc.parallel_loop(0, DIM, unroll=2)
        def _(m):
            m2 = m * 2
            for j in range(nk):
                vals = plsc.load_gather(mid, [s2base[j] + m2])
                dst[m, pl.ds(j * NLANE, NLANE)] = vals

    def stage(t, b, *, wait_o, issue_i):
        wait_in(b)
        if wait_o:
            wait_out(b)
        transpose(b)
        issue_out(t, b)
        if issue_i:
            issue_in(t + RLOOK, (b + RLOOK) % RNBUF)

    for t in range(RLOOK):
        issue_in(t, t)
    for t in range(RNBUF):
        stage(t, t, wait_o=False, issue_i=True)

    def outer(t3, carry):
        for b in range(RNBUF):
            stage(t3 * RNBUF + b, b, wait_o=True, issue_i=True)
        return carry

    lax.fori_loop(1, TBLK // RNBUF - 1, outer, 0)

    for t in range(RNBUF * (TBLK // RNBUF - 1), TBLK):
        stage(t, t % RNBUF, wait_o=True, issue_i=(t + RLOOK < TBLK))
    for b in range(RNBUF):
        wait_out(b)

    @pl.when(wid < VTILES - NW * TBLK)
    def _():
        v0 = (NW * TBLK + wid) * 128
        pltpu.sync_copy(tokt_hbm.at[:, pl.ds(v0, 128)], ibuf[0])
        transpose(0)
        r0 = (NW * TBLK + wid) * DIM
        pltpu.sync_copy(sbuf[0].at[:, pl.ds(0, 128)],
                        out_hbm.at[pl.ds(r0, DIM)])

    @pl.when(wid == NW - 1)
    def _():
        pltpu.sync_copy(tail_hbm, tibuf)
        dst = sbuf[1]

        @plsc.parallel_loop(0, DIM, unroll=2)
        def _(d):
            dp = d * RPITCH
            for k in range(VREM // NLANE):
                vals = tibuf[d, pl.ds(k * NLANE, NLANE)]
                plsc.store_scatter(mid, [s1base[k] + dp], vals)

        @plsc.parallel_loop(0, VREM // 2, unroll=2)
        def _(m):
            m2 = m * 2
            for j in range(128 // NLANE):
                vals = plsc.load_gather(mid, [s2base[j] + m2])
                dst[m, pl.ds(j * NLANE, NLANE)] = vals

        r0 = VTILES * DIM
        pltpu.sync_copy(sbuf[1].at[pl.ds(0, VREM // 2), pl.ds(0, 128)],
                        out_hbm.at[pl.ds(r0, VREM // 2)])


@functools.partial(
    pl.kernel,
    out_type=jax.ShapeDtypeStruct((SENT_LEN, DTILE, NW, 8, 128), jnp.float32),
    mesh=_mesh,
    scratch_types=[
        pltpu.VMEM((NCHUNK, CHUNK), jnp.int32),
        pltpu.VMEM((SENT_LEN, DIM), jnp.float32),
    ]
    + [pltpu.VMEM((CHUNK, DIM), jnp.float32) for _ in range(NBUF)]
    + [pltpu.VMEM((DTILE, 8, 129), jnp.float32) for _ in range(NBUF)]
    + [pltpu.SemaphoreType.DMA for _ in range(2 * NBUF)],
    compiler_params=pltpu.CompilerParams(
        use_tc_tiling_on_sc=False, needs_layout_passes=False),
)
def _emb_lookup(idx_hbm, tok_hbm, pos_hbm, out_hbm, idx_v, pos_v, *bufs_sems):
    gbuf = list(bufs_sems[:NBUF])
    tbuf = list(bufs_sems[NBUF:2 * NBUF])
    gsem = list(bufs_sems[2 * NBUF:3 * NBUF])
    ssem = list(bufs_sems[3 * NBUF:])

    wid = lax.axis_index("s") * 2 + lax.axis_index("c")

    pltpu.sync_copy(idx_hbm.at[:, pl.ds(wid * CHUNK, CHUNK)], idx_v)
    pltpu.sync_copy(pos_hbm, pos_v)

    iota = lax.iota(jnp.int32, NLANE)

    def issue_gather(c, b):
        pltpu.async_copy(tok_hbm.at[idx_v.at[c]], gbuf[b], gsem[b])

    def wait_gather(b):
        pltpu.make_async_copy(tok_hbm.at[pl.ds(0, CHUNK)], gbuf[b], gsem[b]).wait()

    def issue_scatter(c, b):
        pltpu.async_copy(tbuf[b].at[:, :, pl.ds(0, 128)],
                         out_hbm.at[c, :, wid], ssem[b])

    def wait_scatter(b):
        pltpu.make_async_copy(tbuf[b].at[:, :, pl.ds(0, 128)],
                              out_hbm.at[0, :, 0], ssem[b]).wait()

    dr_vecs = [lax.shift_right_logical(iota + j * NLANE, 3)
               for j in range(DIM // NLANE)]
    di_vecs = [lax.bitwise_and(iota + j * NLANE, 7)
               for j in range(DIM // NLANE)]

    def transpose_add(c, b):
        src = gbuf[b]
        dst = tbuf[b]
        posv = [pos_v[c, pl.ds(j * NLANE, NLANE)] for j in range(DIM // NLANE)]

        @plsc.parallel_loop(0, CHUNK, unroll=2)
        def _(bb):
            bsplat = jnp.full((NLANE,), bb, jnp.int32)
            for j in range(DIM // NLANE):
                vals = src[bb, pl.ds(j * NLANE, NLANE)] + posv[j]
                plsc.store_scatter(dst, [dr_vecs[j], di_vecs[j], bsplat], vals)

    def stage(c, b, *, wait_sc, issue_g):
        wait_gather(b)
        if wait_sc:
            wait_scatter(b)
        transpose_add(c, b)
        issue_scatter(c, b)
        if issue_g:
            issue_gather(c + LOOKAHEAD, (b + LOOKAHEAD) % NBUF)

    for c in range(LOOKAHEAD):
        issue_gather(c, c)

    for b in range(NBUF):
        stage(b, b, wait_sc=False, issue_g=True)

    def outer(c4, carry):
        for b in range(NBUF):
            stage(c4 * NBUF + b, b, wait_sc=True, issue_g=True)
        return carry

    lax.fori_loop(1, NCHUNK // NBUF - 1, outer, 0)

    for b in range(NBUF):
        c = NCHUNK - NBUF + b
        stage(c, b, wait_sc=True, issue_g=(c + LOOKAHEAD < NCHUNK))

    for b in range(NBUF):
        wait_scatter(b)


def kernel(inputs, token_table, position_table):
    idx_t = inputs.T.astype(jnp.int32)
    tok_rm = _table_repack(
        token_table.T, token_table[VTILES * 128:].T).reshape(VOCAB, DIM)
    out5 = _emb_lookup(idx_t, tok_rm, position_table)
    return out5.transpose(2, 4, 0, 1, 3).reshape(BATCH, SENT_LEN, DIM)
